# baseline (speedup 1.0000x reference)
import functools

import jax
import jax.numpy as jnp
from jax import lax
from jax.experimental import pallas as pl
from jax.experimental.pallas import tpu as pltpu

_B, _C, _N, _K, _OUT = 16, 64, 1024, 20, 64
_RB = 1024

_NATIVE = (((1,), (0,)), ((), ()))


def _edgeconv_body(xb_ref, xbt_ref, xr_ref, xrt_ref, w1dt_ref, w1bt_ref,
                   b1_ref, w2t_ref, b2_ref, out_ref, cur_ref):
    xb = xb_ref[0]
    xr = xr_ref[0]
    sqb = jnp.sum(xb * xb, axis=1, keepdims=True)
    sqr = jnp.sum(xr * xr, axis=1, keepdims=True).T
    dott = lax.dot_general(xb, xrt_ref[0], _NATIVE,
                           preferred_element_type=jnp.float32)
    cur_ref[...] = (sqr - 2.0 * dott) + sqb

    vt = lax.dot_general(w1bt_ref[...], xbt_ref[0], _NATIVE,
                         preferred_element_type=jnp.float32)
    ut = lax.dot_general(w1dt_ref[...], xrt_ref[0], _NATIVE,
                         preferred_element_type=jnp.float32)
    ubt = ut + b1_ref[...]
    w2t = w2t_ref[...]
    b2 = b2_ref[...]

    iota = lax.broadcasted_iota(jnp.int32, (_N, _RB), 0)
    neg = jnp.full((_OUT, _RB), -jnp.inf, jnp.float32)

    def scan_once():
        cur = cur_ref[...]
        v, ix = cur, iota
        n = _N
        while n > 1:
            h = n // 2
            v1, v2 = v[:h], v[h:]
            take = v2 < v1
            v = jnp.where(take, v2, v1)
            ix = jnp.where(take, ix[h:], ix[:h])
            n = h
        amin = ix
        onehot = iota == amin
        cur_ref[...] = jnp.where(onehot, jnp.inf, cur)
        return lax.dot_general(vt, jnp.where(onehot, 1.0, 0.0), _NATIVE,
                               preferred_element_type=jnp.float32)

    def mlp(vjt):
        e = jnp.maximum(ubt + vjt, 0.0)
        return jnp.maximum(
            lax.dot_general(w2t, e, _NATIVE,
                            preferred_element_type=jnp.float32) + b2, 0.0)

    acc = neg
    for _ in range(_K):
        acc = jnp.maximum(acc, mlp(scan_once()))
    out_ref[0] = acc


@functools.partial(jax.jit, static_argnames=("interpret",))
def kernel(x, W1, b1, W2, b2, interpret=False):
    xt = x[..., 0]
    xf = jnp.transpose(xt, (0, 2, 1))
    w1a, w1b = W1[:_C], W1[_C:]
    w1dt = (w1a - w1b).T
    w1bt = w1b.T

    grid = (_B, _N // _RB)
    out = pl.pallas_call(
        _edgeconv_body,
        grid=grid,
        in_specs=[
            pl.BlockSpec((1, _N, _C), lambda b, r: (b, 0, 0)),
            pl.BlockSpec((1, _C, _N), lambda b, r: (b, 0, 0)),
            pl.BlockSpec((1, _RB, _C), lambda b, r: (b, r, 0)),
            pl.BlockSpec((1, _C, _RB), lambda b, r: (b, 0, r)),
            pl.BlockSpec((_OUT, _C), lambda b, r: (0, 0)),
            pl.BlockSpec((_OUT, _C), lambda b, r: (0, 0)),
            pl.BlockSpec((_OUT, 1), lambda b, r: (0, 0)),
            pl.BlockSpec((_OUT, _OUT), lambda b, r: (0, 0)),
            pl.BlockSpec((_OUT, 1), lambda b, r: (0, 0)),
        ],
        out_specs=pl.BlockSpec((1, _OUT, _RB), lambda b, r: (b, 0, r)),
        out_shape=jax.ShapeDtypeStruct((_B, _OUT, _N), jnp.float32),
        scratch_shapes=[pltpu.VMEM((_N, _RB), jnp.float32)],
        interpret=interpret,
    )(xf, xt, xf, xt, w1dt, w1bt, b1[:, None], W2.T, b2[:, None])
    return out[..., None]


# speedup vs baseline: 1.0045x; 1.0045x over previous
import functools

import jax
import jax.numpy as jnp
from jax import lax
from jax.experimental import pallas as pl
from jax.experimental.pallas import tpu as pltpu

_B, _C, _N, _K, _OUT = 16, 64, 1024, 20, 64
_RB = 1024

_NATIVE = (((1,), (0,)), ((), ()))


def _edgeconv_body(xb_ref, xbt_ref, xr_ref, xrt_ref, w1dt_ref, w1bt_ref,
                   b1_ref, w2t_ref, b2_ref, out_ref, cur_ref):
    xb = xb_ref[0]
    xr = xr_ref[0]
    sqb = jnp.sum(xb * xb, axis=1, keepdims=True)
    sqr = jnp.sum(xr * xr, axis=1, keepdims=True).T
    dott = lax.dot_general(xb, xrt_ref[0], _NATIVE,
                           preferred_element_type=jnp.float32)
    cur_ref[...] = (sqr - 2.0 * dott) + sqb

    vt = lax.dot_general(w1bt_ref[...], xbt_ref[0], _NATIVE,
                         preferred_element_type=jnp.float32)
    ut = lax.dot_general(w1dt_ref[...], xrt_ref[0], _NATIVE,
                         preferred_element_type=jnp.float32)
    ubt = ut + b1_ref[...]
    w2t = w2t_ref[...]
    b2 = b2_ref[...]

    iota = lax.broadcasted_iota(jnp.int32, (_N, _RB), 0)
    neg = jnp.full((_OUT, _RB), -jnp.inf, jnp.float32)

    def scan_once(pending):
        cur = cur_ref[...]
        if pending is not None:
            cur = jnp.where(pending, jnp.inf, cur)
            cur_ref[...] = cur
        v, ix = cur, iota
        n = _N
        while n > 1:
            h = n // 2
            v1, v2 = v[:h], v[h:]
            take = v2 < v1
            v = jnp.where(take, v2, v1)
            ix = jnp.where(take, ix[h:], ix[:h])
            n = h
        onehot = iota == ix
        vjt = lax.dot_general(vt, jnp.where(onehot, 1.0, 0.0), _NATIVE,
                              preferred_element_type=jnp.float32)
        return onehot, vjt

    def mlp(vjt):
        e = jnp.maximum(ubt + vjt, 0.0)
        return jnp.maximum(
            lax.dot_general(w2t, e, _NATIVE,
                            preferred_element_type=jnp.float32) + b2, 0.0)

    acc = neg
    pending = None
    for _ in range(_K):
        pending, vjt = scan_once(pending)
        acc = jnp.maximum(acc, mlp(vjt))
    out_ref[0] = acc


@functools.partial(jax.jit, static_argnames=("interpret",))
def kernel(x, W1, b1, W2, b2, interpret=False):
    xt = x[..., 0]
    xf = jnp.transpose(xt, (0, 2, 1))
    w1a, w1b = W1[:_C], W1[_C:]
    w1dt = (w1a - w1b).T
    w1bt = w1b.T

    grid = (_B, _N // _RB)
    out = pl.pallas_call(
        _edgeconv_body,
        grid=grid,
        in_specs=[
            pl.BlockSpec((1, _N, _C), lambda b, r: (b, 0, 0)),
            pl.BlockSpec((1, _C, _N), lambda b, r: (b, 0, 0)),
            pl.BlockSpec((1, _RB, _C), lambda b, r: (b, r, 0)),
            pl.BlockSpec((1, _C, _RB), lambda b, r: (b, 0, r)),
            pl.BlockSpec((_OUT, _C), lambda b, r: (0, 0)),
            pl.BlockSpec((_OUT, _C), lambda b, r: (0, 0)),
            pl.BlockSpec((_OUT, 1), lambda b, r: (0, 0)),
            pl.BlockSpec((_OUT, _OUT), lambda b, r: (0, 0)),
            pl.BlockSpec((_OUT, 1), lambda b, r: (0, 0)),
        ],
        out_specs=pl.BlockSpec((1, _OUT, _RB), lambda b, r: (b, 0, r)),
        out_shape=jax.ShapeDtypeStruct((_B, _OUT, _N), jnp.float32),
        scratch_shapes=[pltpu.VMEM((_N, _RB), jnp.float32)],
        interpret=interpret,
    )(xf, xt, xf, xt, w1dt, w1bt, b1[:, None], W2.T, b2[:, None])
    return out[..., None]
